# baseline (speedup 1.0000x reference)
import jax
import jax.numpy as jnp
from jax import lax
from jax.experimental import pallas as pl
from jax.experimental.pallas import tpu as pltpu
from jax.experimental.pallas import tpu_sc as plsc

EMB = 32
NC = 2
NS = 16
NW = NC * NS
TCHUNK = 800
BC = 512


def _worker_id():
    return lax.axis_index("s") * NC + lax.axis_index("c")


def _transpose_block(in_v, out_v, n_rows):
    rows_lo = lax.iota(jnp.int32, 16)
    rows_hi = rows_lo + 16
    unroll = 8

    def vbody(jj, carry):
        base = jj * unroll
        for u in range(unroll):
            v = base + u
            col = jnp.full((16,), v, jnp.int32)
            out_v[v, pl.ds(0, 16)] = plsc.load_gather(in_v, [rows_lo, col])
            out_v[v, pl.ds(16, 16)] = plsc.load_gather(in_v, [rows_hi, col])
        return carry

    lax.fori_loop(0, n_rows // unroll, vbody, 0)


def _table_body(tableT_hbm, table_rm_hbm, in0, in1, out0, out1,
                si0, si1, so0, so1):
    n_vocab = tableT_hbm.shape[1]
    n_chunks = n_vocab // TCHUNK
    wid = _worker_id()
    niter = (n_chunks + NW - 1) // NW

    slots = ((in0, out0, si0, so0), (in1, out1, si1, so1))

    def issue_in(cid, s):
        in_v, _, si, _ = slots[s]
        pltpu.async_copy(tableT_hbm.at[:, pl.ds(cid * TCHUNK, TCHUNK)], in_v, si)

    def wait_in(s):
        in_v, _, si, _ = slots[s]
        pltpu.make_async_copy(
            tableT_hbm.at[:, pl.ds(0, TCHUNK)], in_v, si).wait()

    def issue_out(cid, s):
        _, out_v, _, so = slots[s]
        pltpu.async_copy(out_v, table_rm_hbm.at[pl.ds(cid * TCHUNK, TCHUNK)], so)

    def wait_out(s):
        _, out_v, _, so = slots[s]
        pltpu.make_async_copy(
            out_v, table_rm_hbm.at[pl.ds(0, TCHUNK)], so).wait()

    issue_in(wid, 0)

    def pair_body(k, carry):
        for b in range(2):
            i = 2 * k + b
            cid = wid + i * NW
            nxt = wid + (i + 1) * NW

            @pl.when(nxt < n_chunks)
            def _():
                issue_in(nxt, 1 - b)

            @pl.when(cid < n_chunks)
            def _():
                wait_in(b)

                @pl.when(i >= 2)
                def _():
                    wait_out(b)

                _transpose_block(slots[b][0], slots[b][1], TCHUNK)
                issue_out(cid, b)
        return carry

    lax.fori_loop(0, niter // 2, pair_body, 0)
    wait_out(0)
    wait_out(1)


def _gather_body(xT_hbm, table_rm_hbm, posT_hbm, out_hbm,
                 idx0, idx1, buf0, buf1, ob0, ob1, pos_v,
                 sg0, sg1, so0, so1):
    seq = xT_hbm.shape[0]
    batch = xT_hbm.shape[1]
    nh = batch // BC
    n_tasks = seq * nh
    wid = _worker_id()
    niter = n_tasks // NW

    slots = ((idx0, buf0, ob0, sg0, so0), (idx1, buf1, ob1, sg1, so1))

    pltpu.sync_copy(posT_hbm, pos_v)

    def issue_gather(t, s):
        idx_r, buf_r, _, gs, _ = slots[s]
        l = t // nh
        b0 = (t % nh) * BC
        pltpu.sync_copy(xT_hbm.at[l, pl.ds(b0, BC)], idx_r)
        pltpu.async_copy(table_rm_hbm.at[idx_r], buf_r, gs)

    def wait_gather(s):
        idx_r, buf_r, _, gs, _ = slots[s]
        pltpu.make_async_copy(table_rm_hbm.at[idx_r], buf_r, gs).wait()

    def issue_out(t, s):
        _, _, ob_r, _, so = slots[s]
        l = t // nh
        bt0 = (t % nh) * (BC // 128)
        pltpu.async_copy(ob_r, out_hbm.at[l, :, pl.ds(bt0, BC // 128), :, :], so)

    def wait_out(s):
        _, _, ob_r, _, so = slots[s]
        pltpu.make_async_copy(
            ob_r, out_hbm.at[0, :, pl.ds(0, BC // 128), :, :], so).wait()

    def transpose_add(buf_r, ob_r, l):
        rows = lax.iota(jnp.int32, 16)
        unroll = 8
        l16 = jnp.full((16,), l, jnp.int32)

        def ebody(e, carry):
            e16 = jnp.full((16,), e, jnp.int32)
            pv = plsc.load_gather(pos_v, [e16, l16])
            et = e // 8
            ei = e % 8

            def vbody(jj, carry2):
                base = jj * unroll
                for u in range(unroll):
                    j = base + u
                    r = rows + j * 16
                    btr = j // 8
                    bi0 = (j % 8) * 16
                    ob_r[et, btr, ei, pl.ds(bi0, 16)] = (
                        plsc.load_gather(buf_r, [r, e16]) + pv)
                return carry2

            lax.fori_loop(0, BC // (16 * unroll), vbody, 0)
            return carry

        lax.fori_loop(0, EMB, ebody, 0)

    issue_gather(wid, 0)

    def pair_body(k, carry):
        for b in range(2):
            i = 2 * k + b
            t = wid + i * NW

            @pl.when(i + 1 < niter)
            def _():
                issue_gather(wid + (i + 1) * NW, 1 - b)

            wait_gather(b)

            @pl.when(i >= 2)
            def _():
                wait_out(b)

            transpose_add(slots[b][1], slots[b][2], t // nh)
            issue_out(t, b)
        return carry

    lax.fori_loop(0, niter // 2, pair_body, 0)
    wait_out(0)
    wait_out(1)


def _make_calls(vocab, seq, batch, max_pos):
    mesh = plsc.VectorSubcoreMesh(core_axis_name="c", subcore_axis_name="s")
    table_call = pl.kernel(
        _table_body,
        out_type=jax.ShapeDtypeStruct((vocab, EMB), jnp.float32),
        mesh=mesh,
        scratch_types=[
            pltpu.VMEM((EMB, TCHUNK), jnp.float32),
            pltpu.VMEM((EMB, TCHUNK), jnp.float32),
            pltpu.VMEM((TCHUNK, EMB), jnp.float32),
            pltpu.VMEM((TCHUNK, EMB), jnp.float32),
            pltpu.SemaphoreType.DMA,
            pltpu.SemaphoreType.DMA,
            pltpu.SemaphoreType.DMA,
            pltpu.SemaphoreType.DMA,
        ],
        compiler_params=pltpu.CompilerParams(use_tc_tiling_on_sc=False, needs_layout_passes=False),
    )
    gather_call = pl.kernel(
        _gather_body,
        out_type=jax.ShapeDtypeStruct((seq, EMB // 8, batch // 128, 8, 128),
                                      jnp.float32),
        mesh=mesh,
        scratch_types=[
            pltpu.VMEM((BC,), jnp.int32),
            pltpu.VMEM((BC,), jnp.int32),
            pltpu.VMEM((BC, EMB), jnp.float32),
            pltpu.VMEM((BC, EMB), jnp.float32),
            pltpu.VMEM((EMB // 8, BC // 128, 8, 128), jnp.float32),
            pltpu.VMEM((EMB // 8, BC // 128, 8, 128), jnp.float32),
            pltpu.VMEM((EMB, max_pos), jnp.float32),
            pltpu.SemaphoreType.DMA,
            pltpu.SemaphoreType.DMA,
            pltpu.SemaphoreType.DMA,
            pltpu.SemaphoreType.DMA,
        ],
        compiler_params=pltpu.CompilerParams(use_tc_tiling_on_sc=False, needs_layout_passes=False),
    )
    return table_call, gather_call


def kernel(x, token_table, pos_table):
    batch, seq = x.shape
    vocab = token_table.shape[0]
    max_pos = pos_table.shape[0]
    xT = jnp.transpose(x).astype(jnp.int32)
    tableT = jnp.transpose(token_table)
    posT = jnp.transpose(pos_table)
    table_call, gather_call = _make_calls(vocab, seq, batch, max_pos)
    table_rm = table_call(tableT)
    out5 = gather_call(xT, table_rm, posT)
    out = jnp.transpose(out5, (2, 4, 0, 1, 3))
    return out.reshape(batch, seq, EMB)


# speedup vs baseline: 3.8713x; 3.8713x over previous
import jax
import jax.numpy as jnp
from jax import lax
from jax.experimental import pallas as pl
from jax.experimental.pallas import tpu as pltpu
from jax.experimental.pallas import tpu_sc as plsc

EMB = 32
NC = 2
NS = 16
NW = NC * NS
BC = 512


def _worker_id():
    return lax.axis_index("s") * NC + lax.axis_index("c")


def _gather_body(xT_hbm, table_hbm, posT_hbm, out_hbm,
                 idx0, idx1, buf0, buf1, ob0, ob1, pos_v,
                 sg0, sg1, so0, so1):
    seq = xT_hbm.shape[0]
    batch = xT_hbm.shape[1]
    nh = batch // BC
    n_tasks = seq * nh
    wid = _worker_id()
    niter = n_tasks // NW

    slots = ((idx0, buf0, ob0, sg0, so0), (idx1, buf1, ob1, sg1, so1))

    pltpu.sync_copy(posT_hbm, pos_v)

    def issue_gather(t, s):
        idx_r, buf_r, _, gs, _ = slots[s]
        l = t // nh
        b0 = (t % nh) * BC
        pltpu.sync_copy(xT_hbm.at[l, pl.ds(b0, BC)], idx_r)
        pltpu.async_copy(table_hbm.at[idx_r], buf_r, gs)

    def wait_gather(s):
        idx_r, buf_r, _, gs, _ = slots[s]
        pltpu.make_async_copy(table_hbm.at[idx_r], buf_r, gs).wait()

    def issue_out(t, s):
        _, _, ob_r, _, so = slots[s]
        l = t // nh
        bt0 = (t % nh) * (BC // 128)
        pltpu.async_copy(ob_r, out_hbm.at[l, :, pl.ds(bt0, BC // 128), :, :], so)

    def wait_out(s):
        _, _, ob_r, _, so = slots[s]
        pltpu.make_async_copy(
            ob_r, out_hbm.at[0, :, pl.ds(0, BC // 128), :, :], so).wait()

    def transpose_add(buf_r, ob_r, l):
        rows = lax.iota(jnp.int32, 16)
        unroll = 8
        l16 = jnp.full((16,), l, jnp.int32)

        def ebody(e, carry):
            e16 = jnp.full((16,), e, jnp.int32)
            pv = plsc.load_gather(pos_v, [e16, l16])
            et = e // 8
            ei = e % 8

            def vbody(jj, row0):
                vals = []
                for u in range(unroll):
                    r = row0 + u * 16
                    vals.append(plsc.load_gather(buf_r, [r, e16]) + pv)
                for u, val in enumerate(vals):
                    ob_r[et, jj, ei, pl.ds(u * 16, 16)] = val
                return row0 + unroll * 16

            lax.fori_loop(0, BC // (16 * unroll), vbody, rows)
            return carry

        lax.fori_loop(0, EMB, ebody, 0)

    issue_gather(wid, 0)

    def pair_body(k, carry):
        for b in range(2):
            i = 2 * k + b
            t = wid + i * NW

            @pl.when(i + 1 < niter)
            def _():
                issue_gather(wid + (i + 1) * NW, 1 - b)

            wait_gather(b)

            @pl.when(i >= 2)
            def _():
                wait_out(b)

            transpose_add(slots[b][1], slots[b][2], t // nh)
            issue_out(t, b)
        return carry

    lax.fori_loop(0, niter // 2, pair_body, 0)
    wait_out(0)
    wait_out(1)


def _make_call(vocab, seq, batch, max_pos):
    mesh = plsc.VectorSubcoreMesh(core_axis_name="c", subcore_axis_name="s")
    return pl.kernel(
        _gather_body,
        out_type=jax.ShapeDtypeStruct((seq, EMB // 8, batch // 128, 8, 128),
                                      jnp.float32),
        mesh=mesh,
        scratch_types=[
            pltpu.VMEM((BC,), jnp.int32),
            pltpu.VMEM((BC,), jnp.int32),
            pltpu.VMEM((BC, EMB), jnp.float32),
            pltpu.VMEM((BC, EMB), jnp.float32),
            pltpu.VMEM((EMB // 8, BC // 128, 8, 128), jnp.float32),
            pltpu.VMEM((EMB // 8, BC // 128, 8, 128), jnp.float32),
            pltpu.VMEM((EMB, max_pos), jnp.float32),
            pltpu.SemaphoreType.DMA,
            pltpu.SemaphoreType.DMA,
            pltpu.SemaphoreType.DMA,
            pltpu.SemaphoreType.DMA,
        ],
        compiler_params=pltpu.CompilerParams(use_tc_tiling_on_sc=False,
                                             needs_layout_passes=False),
    )


def kernel(x, token_table, pos_table):
    batch, seq = x.shape
    vocab = token_table.shape[0]
    max_pos = pos_table.shape[0]
    xT = jnp.transpose(x).astype(jnp.int32)
    posT = jnp.transpose(pos_table)
    out5 = _make_call(vocab, seq, batch, max_pos)(xT, token_table, posT)
    out = jnp.transpose(out5, (2, 4, 0, 1, 3))
    return out.reshape(batch, seq, EMB)


# speedup vs baseline: 3.9648x; 1.0242x over previous
import jax
import jax.numpy as jnp
from jax import lax
from jax.experimental import pallas as pl
from jax.experimental.pallas import tpu as pltpu
from jax.experimental.pallas import tpu_sc as plsc

EMB = 32
NC = 2
NS = 16
NW = NC * NS
BC = 512


def _worker_id():
    return lax.axis_index("s") * NC + lax.axis_index("c")


def _gather_body(x_hbm, table_hbm, posT_hbm, out_hbm,
                 idx_all, buf0, buf1, ob0, ob1, pos_v,
                 sg0, sg1, so0, so1):
    n_rows = x_hbm.shape[0]
    batch = out_hbm.shape[2] * 128
    nh = batch // BC
    wid = _worker_id()
    rows_per_worker = n_rows // NW
    niter = rows_per_worker // BC

    slots = ((buf0, ob0, sg0, so0), (buf1, ob1, sg1, so1))

    pltpu.sync_copy(posT_hbm, pos_v)
    pltpu.sync_copy(x_hbm.at[pl.ds(wid * rows_per_worker, rows_per_worker)],
                    idx_all)

    def issue_gather(i, s):
        buf_r, _, gs, _ = slots[s]
        pltpu.async_copy(
            table_hbm.at[idx_all.at[pl.ds(i * BC, BC)]], buf_r, gs)

    def wait_gather(s):
        buf_r, _, gs, _ = slots[s]
        pltpu.make_async_copy(
            table_hbm.at[idx_all.at[pl.ds(0, BC)]], buf_r, gs).wait()

    def issue_out(t, s):
        _, ob_r, _, so = slots[s]
        l = t // nh
        bt0 = (t % nh) * (BC // 128)
        pltpu.async_copy(ob_r, out_hbm.at[l, :, pl.ds(bt0, BC // 128), :, :], so)

    def wait_out(s):
        _, ob_r, _, so = slots[s]
        pltpu.make_async_copy(
            ob_r, out_hbm.at[0, :, pl.ds(0, BC // 128), :, :], so).wait()

    def transpose_add(buf_r, ob_r, l):
        rows = lax.iota(jnp.int32, 16)
        unroll = 8
        l16 = jnp.full((16,), l, jnp.int32)

        def ebody(e, carry):
            e16 = jnp.full((16,), e, jnp.int32)
            pv = plsc.load_gather(pos_v, [e16, l16])
            et = e // 8
            ei = e % 8

            def vbody(jj, row0):
                vals = []
                for u in range(unroll):
                    r = row0 + u * 16
                    vals.append(plsc.load_gather(buf_r, [r, e16]) + pv)
                for u, val in enumerate(vals):
                    ob_r[et, jj, ei, pl.ds(u * 16, 16)] = val
                return row0 + unroll * 16

            lax.fori_loop(0, BC // (16 * unroll), vbody, rows)
            return carry

        lax.fori_loop(0, EMB, ebody, 0)

    issue_gather(0, 0)

    def pair_body(k, carry):
        for b in range(2):
            i = 2 * k + b
            t = wid * niter + i

            @pl.when(i + 1 < niter)
            def _():
                issue_gather(i + 1, 1 - b)

            wait_gather(b)

            @pl.when(i >= 2)
            def _():
                wait_out(b)

            transpose_add(slots[b][0], slots[b][1], t // nh)
            issue_out(t, b)
        return carry

    lax.fori_loop(0, niter // 2, pair_body, 0)
    wait_out(0)
    wait_out(1)


def _make_call(vocab, seq, batch, max_pos):
    mesh = plsc.VectorSubcoreMesh(core_axis_name="c", subcore_axis_name="s")
    return pl.kernel(
        _gather_body,
        out_type=jax.ShapeDtypeStruct((seq, EMB // 8, batch // 128, 8, 128),
                                      jnp.float32),
        mesh=mesh,
        scratch_types=[
            pltpu.VMEM(((seq * batch) // NW,), jnp.int32),
            pltpu.VMEM((BC, EMB), jnp.float32),
            pltpu.VMEM((BC, EMB), jnp.float32),
            pltpu.VMEM((EMB // 8, BC // 128, 8, 128), jnp.float32),
            pltpu.VMEM((EMB // 8, BC // 128, 8, 128), jnp.float32),
            pltpu.VMEM((EMB, max_pos), jnp.float32),
            pltpu.SemaphoreType.DMA,
            pltpu.SemaphoreType.DMA,
            pltpu.SemaphoreType.DMA,
            pltpu.SemaphoreType.DMA,
        ],
        compiler_params=pltpu.CompilerParams(use_tc_tiling_on_sc=False,
                                             needs_layout_passes=False),
    )


def kernel(x, token_table, pos_table):
    batch, seq = x.shape
    vocab = token_table.shape[0]
    max_pos = pos_table.shape[0]
    x_flat = jnp.transpose(x).astype(jnp.int32).reshape(seq * batch)
    posT = jnp.transpose(pos_table)
    out5 = _make_call(vocab, seq, batch, max_pos)(x_flat, token_table, posT)
    out = jnp.transpose(out5, (2, 4, 0, 1, 3))
    return out.reshape(batch, seq, EMB)


# speedup vs baseline: 3.9829x; 1.0046x over previous
import jax
import jax.numpy as jnp
from jax import lax
from jax.experimental import pallas as pl
from jax.experimental.pallas import tpu as pltpu
from jax.experimental.pallas import tpu_sc as plsc

EMB = 32
SEQ = 200
NC = 2
NS = 16
NW = NC * NS
CHUNK = 800
REPS = CHUNK // SEQ


def _worker_id():
    return lax.axis_index("s") * NC + lax.axis_index("c")


def _body(x_hbm, table_hbm, pos_hbm, out_hbm,
          idx0, idx1, buf0, buf1, pos_v, gsem0, gsem1, wsem0, wsem1):
    rows_per_worker = x_hbm.shape[0] // NW
    n_chunks = rows_per_worker // CHUNK
    wid = _worker_id()
    base = wid * rows_per_worker

    slots = ((idx0, buf0, gsem0, wsem0), (idx1, buf1, gsem1, wsem1))

    def issue_gather(c, slot):
        idx_r, buf_r, gs, _ = slots[slot]
        start = base + c * CHUNK
        pltpu.sync_copy(x_hbm.at[pl.ds(start, CHUNK)], idx_r)
        pltpu.async_copy(table_hbm.at[idx_r], buf_r, gs)

    def wait_gather(slot):
        idx_r, buf_r, gs, _ = slots[slot]
        pltpu.make_async_copy(table_hbm.at[idx_r], buf_r, gs).wait()

    def issue_wb(c, slot):
        _, buf_r, _, ws = slots[slot]
        start = base + c * CHUNK
        pltpu.async_copy(buf_r, out_hbm.at[pl.ds(start, CHUNK)], ws)

    def wait_wb(slot):
        _, buf_r, _, ws = slots[slot]
        pltpu.make_async_copy(buf_r, out_hbm.at[pl.ds(base, CHUNK)], ws).wait()

    def add_pos(buf_r):
        def body_j(j, carry):
            p0 = pos_v[j, pl.ds(0, 16)]
            p1 = pos_v[j, pl.ds(16, 16)]
            for rep in range(REPS):
                i = rep * SEQ + j
                buf_r[i, pl.ds(0, 16)] += p0
                buf_r[i, pl.ds(16, 16)] += p1
            return carry

        lax.fori_loop(0, SEQ, body_j, 0)

    pltpu.sync_copy(pos_hbm.at[pl.ds(0, SEQ)], pos_v)
    issue_gather(0, 0)

    def pair_body(g, carry):
        for b in range(2):
            c = 2 * g + b

            @pl.when(c + 1 < n_chunks)
            def _():
                @pl.when(c >= 1)
                def _():
                    wait_wb(1 - b)

                issue_gather(c + 1, 1 - b)

            wait_gather(b)
            add_pos(slots[b][1])
            issue_wb(c, b)
        return carry

    lax.fori_loop(0, n_chunks // 2, pair_body, 0)
    wait_wb(0)
    wait_wb(1)


def _make_sc_call(n_rows):
    mesh = plsc.VectorSubcoreMesh(core_axis_name="c", subcore_axis_name="s")
    return pl.kernel(
        _body,
        out_type=jax.ShapeDtypeStruct((n_rows, EMB), jnp.float32),
        mesh=mesh,
        scratch_types=[
            pltpu.VMEM((CHUNK,), jnp.int32),
            pltpu.VMEM((CHUNK,), jnp.int32),
            pltpu.VMEM((CHUNK, EMB), jnp.float32),
            pltpu.VMEM((CHUNK, EMB), jnp.float32),
            pltpu.VMEM((SEQ, EMB), jnp.float32),
            pltpu.SemaphoreType.DMA,
            pltpu.SemaphoreType.DMA,
            pltpu.SemaphoreType.DMA,
            pltpu.SemaphoreType.DMA,
        ],
        compiler_params=pltpu.CompilerParams(use_tc_tiling_on_sc=False),
    )


def kernel(x, token_table, pos_table):
    b, l = x.shape
    n = b * l
    x_flat = x.reshape(n).astype(jnp.int32)
    out = _make_sc_call(n)(x_flat, token_table, pos_table)
    return out.reshape(b, l, EMB)
